# baseline (speedup 1.0000x reference)
import functools

import jax
import jax.numpy as jnp
from jax import lax
from jax.experimental import pallas as pl
from jax.experimental.pallas import tpu as pltpu
from jax.experimental.pallas import tpu_sc as plsc

_K = 20
_R = 256
_CD = 32
_GSUB = 80
_EB = 2560


def _topk_body(din, nblk, refs):
    xf_ref, idx_ref, d2c_ref, d2r_ref = refs
    m_total = nblk * _R
    i = pl.program_id(0)

    @pl.when(i == 0)
    def _():
        xf0 = xf_ref[...]
        sq = xf0[:, :din] * xf0[:, :din] if din & (din - 1) else xf0 * xf0
        w = sq.shape[1]
        if w & (w - 1) == 0:
            while w > 1:
                w //= 2
                sq = sq[:, :w] + sq[:, w:2 * w]
            d2 = sq
        else:
            d2 = jnp.sum(sq, axis=1, keepdims=True)
        d2c_ref[...] = d2
        d2r_ref[...] = lax.transpose(d2, (1, 0))

    xb = xf_ref[pl.ds(i * _R, _R), :]
    xf = xf_ref[...]
    xdot = lax.dot_general(xb, xf, (((1,), (1,)), ((), ())),
                           preferred_element_type=jnp.float32)
    dist = (d2c_ref[pl.ds(i * _R, _R), :] + d2r_ref[...]) - 2.0 * xdot
    col = lax.broadcasted_iota(jnp.int32, (_R, m_total), 1)
    row = i * _R + lax.broadcasted_iota(jnp.int32, (_R, m_total), 0)
    dist = jnp.where(col == row, dist + 1e10, dist)

    picks = []
    for _ in range(_K):
        idxv = jnp.argmin(dist, axis=1).astype(jnp.int32)[:, None]
        dist = jnp.where(col == idxv, jnp.inf, dist)
        picks.append(idxv)
    idx_ref[...] = jnp.concatenate(picks, axis=1)


def _topk(xf, din):
    m = xf.shape[0]
    nblk = m // _R
    return pl.pallas_call(
        lambda *refs: _topk_body(din, nblk, refs),
        grid=(nblk,),
        in_specs=[pl.BlockSpec((m, 128), lambda i: (0, 0))],
        out_specs=pl.BlockSpec((_R, _K), lambda i: (i, 0)),
        out_shape=jax.ShapeDtypeStruct((m, _K), jnp.int32),
        scratch_shapes=[pltpu.VMEM((m, 1), jnp.float32),
                        pltpu.VMEM((1, m), jnp.float32)],
    )(xf)


def _sc_info():
    info = plsc.get_sparse_core_info()
    return info, plsc.VectorSubcoreMesh(core_axis_name="c", subcore_axis_name="s")


def _sc_build_feat(xf, idx_flat, half):
    m = xf.shape[0]
    info, mesh = _sc_info()
    nw = info.num_cores * info.num_subcores
    rows_w = m // nw
    cd = 16
    nchunk = rows_w // cd
    nq = half // 16

    @functools.partial(
        pl.kernel, mesh=mesh,
        out_type=jax.ShapeDtypeStruct((m * _K, 2 * half), jnp.float32),
        scratch_types=[
            pltpu.VMEM((cd * _K,), jnp.int32),
            pltpu.VMEM((cd * _K, 128), jnp.float32),
            pltpu.VMEM((cd, 128), jnp.float32),
            pltpu.VMEM((cd * _K, 2 * half), jnp.float32),
            pltpu.SemaphoreType.DMA,
        ],
    )
    def k(xf_hbm, idx_hbm, feat_hbm, idx_v, rows_v, xi_v, feat_v, sem):
        wid = lax.axis_index("s") * info.num_cores + lax.axis_index("c")
        row0 = wid * rows_w

        def chunk(c, carry):
            base = row0 + c * cd
            pltpu.sync_copy(idx_hbm.at[pl.ds(base * _K, cd * _K)], idx_v)
            copies = [
                pltpu.async_copy(
                    xf_hbm.at[idx_v.at[pl.ds(s * _GSUB, _GSUB)]],
                    rows_v.at[pl.ds(s * _GSUB, _GSUB), :], sem)
                for s in range((cd * _K) // _GSUB)
            ]
            pltpu.sync_copy(xf_hbm.at[pl.ds(base, cd), :], xi_v)
            for cp in copies:
                cp.wait()

            def one_row(r, carry2):
                for q in range(nq):
                    fs = pl.ds(q * 16, 16)
                    xi = xi_v[r, fs]
                    for t in range(_K):
                        e = r * _K + t
                        feat_v[e, fs] = xi
                        feat_v[e, pl.ds(half + q * 16, 16)] = (
                            rows_v[e, fs] - xi)
                return carry2

            lax.fori_loop(0, cd, one_row, 0, unroll=False)
            pltpu.sync_copy(feat_v, feat_hbm.at[pl.ds(base * _K, cd * _K), :])
            return carry

        lax.fori_loop(0, nchunk, chunk, 0, unroll=False)

    return k(xf, idx_flat)


def _edge_matmul(feat, w, b):
    ne, f = feat.shape
    dout = w.shape[1]
    nblk = ne // _EB

    def body(feat_ref, w_ref, b_ref, h_ref):
        h_ref[...] = jnp.dot(feat_ref[...], w_ref[...],
                             preferred_element_type=jnp.float32) + b_ref[...]

    return pl.pallas_call(
        body,
        grid=(nblk,),
        in_specs=[pl.BlockSpec((_EB, f), lambda i: (i, 0)),
                  pl.BlockSpec((f, dout), lambda i: (0, 0)),
                  pl.BlockSpec((1, dout), lambda i: (0, 0))],
        out_specs=pl.BlockSpec((_EB, dout), lambda i: (i, 0)),
        out_shape=jax.ShapeDtypeStruct((ne, dout), jnp.float32),
    )(feat, w, b.reshape(1, dout))


def _sc_segmax(h, m):
    dout = h.shape[1]
    info, mesh = _sc_info()
    nw = info.num_cores * info.num_subcores
    rows_w = m // nw
    nchunk = rows_w // _CD
    nq = dout // 16

    @functools.partial(
        pl.kernel, mesh=mesh,
        out_type=jax.ShapeDtypeStruct((m, 128), jnp.float32),
        scratch_types=[
            pltpu.VMEM((_CD * _K, dout), jnp.float32),
            pltpu.VMEM((_CD, 128), jnp.float32),
            pltpu.SemaphoreType.DMA,
        ],
    )
    def k(h_hbm, out_hbm, rows_v, out_v, sem):
        wid = lax.axis_index("s") * info.num_cores + lax.axis_index("c")
        row0 = wid * rows_w

        def chunk(c, carry):
            base = row0 + c * _CD
            pltpu.sync_copy(h_hbm.at[pl.ds(base * _K, _CD * _K), :], rows_v)

            def one_row(r, carry2):
                for q in range(8):
                    fs = pl.ds(q * 16, 16)
                    if q < nq:
                        acc = rows_v[r * _K, fs]
                        for t in range(1, _K):
                            acc = jnp.maximum(acc, rows_v[r * _K + t, fs])
                        out_v[r, fs] = jnp.maximum(acc, 0.0)
                    else:
                        out_v[r, fs] = jnp.zeros((16,), jnp.float32)
                return carry2

            lax.fori_loop(0, _CD, one_row, 0, unroll=False)
            pltpu.sync_copy(out_v, out_hbm.at[pl.ds(base, _CD), :])
            return carry

        lax.fori_loop(0, nchunk, chunk, 0, unroll=False)

    return k(h)


def _head_body(refs):
    xf_ref, wf1_ref, bf1_ref, wf2_ref, bf2_ref, out_ref = refs
    h = jnp.maximum(jnp.dot(xf_ref[...], wf1_ref[...],
                            preferred_element_type=jnp.float32)
                    + bf1_ref[...], 0.0)
    out_ref[...] = jnp.dot(h, wf2_ref[...],
                           preferred_element_type=jnp.float32) + bf2_ref[...]


def _head(xf, wf1, bf1, wf2, bf2):
    m, d = xf.shape
    full = lambda s: pl.BlockSpec(s, lambda: (0, 0))
    return pl.pallas_call(
        lambda *refs: _head_body(refs),
        in_specs=[full((m, d)), full(wf1.shape), full((1, wf1.shape[1])),
                  full(wf2.shape), full((1, wf2.shape[1]))],
        out_specs=full((m, wf2.shape[1])),
        out_shape=jax.ShapeDtypeStruct((m, wf2.shape[1]), jnp.float32),
    )(xf, wf1, bf1.reshape(1, -1), wf2, bf2.reshape(1, -1))


def _pad_w(w, half, din):
    if half == din:
        return w
    out = jnp.zeros((2 * half, w.shape[1]), jnp.float32)
    out = out.at[:din].set(w[:din])
    out = out.at[half:half + din].set(w[din:])
    return out


def kernel(x, W1, b1, W2, b2, W3, b3, Wf1, bf1, Wf2, bf2):
    B, N, _ = x.shape
    m = B * N
    xf = jnp.pad(x.reshape(m, 3), ((0, 0), (0, 125)))
    for w, b, din, half in ((W1, b1, 3, 16), (W2, b2, 64, 64),
                            (W3, b3, 64, 64)):
        idx = _topk(xf, din)
        feat = _sc_build_feat(xf, idx.reshape(-1), half)
        h = _edge_matmul(feat, _pad_w(w, half, din), b)
        xf = _sc_segmax(h, m)
    out = _head(xf, Wf1, bf1, Wf2, bf2)
    return out.reshape(B, N, 1)


# speedup vs baseline: 1.0034x; 1.0034x over previous
import functools

import jax
import jax.numpy as jnp
from jax import lax
from jax.experimental import pallas as pl
from jax.experimental.pallas import tpu as pltpu
from jax.experimental.pallas import tpu_sc as plsc

_K = 20
_R = 256
_CD = 32
_GSUB = 80
_EB = 2560


def _topk_body(din, nblk, refs):
    xf_ref, idx_ref, d2c_ref, d2r_ref = refs
    m_total = nblk * _R
    i = pl.program_id(0)

    @pl.when(i == 0)
    def _():
        xf0 = xf_ref[...]
        sq = xf0[:, :din] * xf0[:, :din] if din & (din - 1) else xf0 * xf0
        w = sq.shape[1]
        if w & (w - 1) == 0:
            while w > 1:
                w //= 2
                sq = sq[:, :w] + sq[:, w:2 * w]
            d2 = sq
        else:
            d2 = jnp.sum(sq, axis=1, keepdims=True)
        d2c_ref[...] = d2
        d2r_ref[...] = lax.transpose(d2, (1, 0))

    xb = xf_ref[pl.ds(i * _R, _R), :]
    xf = xf_ref[...]
    xdot = lax.dot_general(xb, xf, (((1,), (1,)), ((), ())),
                           preferred_element_type=jnp.float32)
    dist = (d2c_ref[pl.ds(i * _R, _R), :] + d2r_ref[...]) - 2.0 * xdot
    col = lax.broadcasted_iota(jnp.int32, (_R, m_total), 1)
    row = i * _R + lax.broadcasted_iota(jnp.int32, (_R, m_total), 0)
    dist = jnp.where(col == row, dist + 1e10, dist)

    picks = []
    for _ in range(_K):
        idxv = jnp.argmin(dist, axis=1).astype(jnp.int32)[:, None]
        dist = jnp.where(col == idxv, jnp.inf, dist)
        picks.append(idxv)
    idx_ref[...] = jnp.concatenate(picks, axis=1)


def _topk(xf, din):
    m = xf.shape[0]
    nblk = m // _R
    return pl.pallas_call(
        lambda *refs: _topk_body(din, nblk, refs),
        grid=(nblk,),
        in_specs=[pl.BlockSpec((m, 128), lambda i: (0, 0))],
        out_specs=pl.BlockSpec((_R, _K), lambda i: (i, 0)),
        out_shape=jax.ShapeDtypeStruct((m, _K), jnp.int32),
        scratch_shapes=[pltpu.VMEM((m, 1), jnp.float32),
                        pltpu.VMEM((1, m), jnp.float32)],
    )(xf)


def _sc_info():
    info = plsc.get_sparse_core_info()
    return info, plsc.VectorSubcoreMesh(core_axis_name="c", subcore_axis_name="s")


def _sc_build_diff(xf, idx_flat, half):
    m = xf.shape[0]
    info, mesh = _sc_info()
    nw = info.num_cores * info.num_subcores
    rows_w = m // nw
    cd = 16
    nchunk = rows_w // cd
    nq = half // 16

    @functools.partial(
        pl.kernel, mesh=mesh,
        out_type=jax.ShapeDtypeStruct((m * _K, half), jnp.float32),
        scratch_types=[
            pltpu.VMEM((cd * _K,), jnp.int32),
            pltpu.VMEM((cd * _K, 128), jnp.float32),
            pltpu.VMEM((cd, 128), jnp.float32),
            pltpu.VMEM((cd * _K, half), jnp.float32),
            pltpu.SemaphoreType.DMA,
        ],
    )
    def k(xf_hbm, idx_hbm, feat_hbm, idx_v, rows_v, xi_v, feat_v, sem):
        wid = lax.axis_index("s") * info.num_cores + lax.axis_index("c")
        row0 = wid * rows_w

        def chunk(c, carry):
            base = row0 + c * cd
            pltpu.sync_copy(idx_hbm.at[pl.ds(base * _K, cd * _K)], idx_v)
            copies = [
                pltpu.async_copy(
                    xf_hbm.at[idx_v.at[pl.ds(s * _GSUB, _GSUB)]],
                    rows_v.at[pl.ds(s * _GSUB, _GSUB), :], sem)
                for s in range((cd * _K) // _GSUB)
            ]
            pltpu.sync_copy(xf_hbm.at[pl.ds(base, cd), :], xi_v)
            for cp in copies:
                cp.wait()

            def one_row(r, carry2):
                for q in range(nq):
                    fs = pl.ds(q * 16, 16)
                    xi = xi_v[r, fs]
                    for t in range(_K):
                        e = r * _K + t
                        feat_v[e, fs] = rows_v[e, fs] - xi
                return carry2

            lax.fori_loop(0, cd, one_row, 0, unroll=False)
            pltpu.sync_copy(feat_v, feat_hbm.at[pl.ds(base * _K, cd * _K), :])
            return carry

        lax.fori_loop(0, nchunk, chunk, 0, unroll=False)

    return k(xf, idx_flat)


def _edge_matmul(diff, xf, w, b, half):
    ne = diff.shape[0]
    dout = w.shape[1]
    nblk = ne // _EB
    rb = _EB // _K

    def body(diff_ref, xf_ref, w_ref, b_ref, h_ref):
        i = pl.program_id(0)
        xb = xf_ref[pl.ds(i * rb, rb), :half]
        xi_rep = jnp.broadcast_to(xb[:, None, :], (rb, _K, half))
        xi_rep = xi_rep.reshape(_EB, half)
        feat = jnp.concatenate([xi_rep, diff_ref[...]], axis=1)
        h_ref[...] = jnp.dot(feat, w_ref[...],
                             preferred_element_type=jnp.float32) + b_ref[...]

    return pl.pallas_call(
        body,
        grid=(nblk,),
        in_specs=[pl.BlockSpec((_EB, half), lambda i: (i, 0)),
                  pl.BlockSpec(xf.shape, lambda i: (0, 0)),
                  pl.BlockSpec((2 * half, dout), lambda i: (0, 0)),
                  pl.BlockSpec((1, dout), lambda i: (0, 0))],
        out_specs=pl.BlockSpec((_EB, dout), lambda i: (i, 0)),
        out_shape=jax.ShapeDtypeStruct((ne, dout), jnp.float32),
    )(diff, xf, w, b.reshape(1, dout))


def _sc_segmax(h, m):
    dout = h.shape[1]
    info, mesh = _sc_info()
    nw = info.num_cores * info.num_subcores
    rows_w = m // nw
    nchunk = rows_w // _CD
    nq = dout // 16

    @functools.partial(
        pl.kernel, mesh=mesh,
        out_type=jax.ShapeDtypeStruct((m, 128), jnp.float32),
        scratch_types=[
            pltpu.VMEM((_CD * _K, dout), jnp.float32),
            pltpu.VMEM((_CD, 128), jnp.float32),
            pltpu.SemaphoreType.DMA,
        ],
    )
    def k(h_hbm, out_hbm, rows_v, out_v, sem):
        wid = lax.axis_index("s") * info.num_cores + lax.axis_index("c")
        row0 = wid * rows_w

        def chunk(c, carry):
            base = row0 + c * _CD
            pltpu.sync_copy(h_hbm.at[pl.ds(base * _K, _CD * _K), :], rows_v)

            def one_row(r, carry2):
                for q in range(8):
                    fs = pl.ds(q * 16, 16)
                    if q < nq:
                        acc = rows_v[r * _K, fs]
                        for t in range(1, _K):
                            acc = jnp.maximum(acc, rows_v[r * _K + t, fs])
                        out_v[r, fs] = jnp.maximum(acc, 0.0)
                    else:
                        out_v[r, fs] = jnp.zeros((16,), jnp.float32)
                return carry2

            lax.fori_loop(0, _CD, one_row, 0, unroll=False)
            pltpu.sync_copy(out_v, out_hbm.at[pl.ds(base, _CD), :])
            return carry

        lax.fori_loop(0, nchunk, chunk, 0, unroll=False)

    return k(h)


def _head_body(refs):
    xf_ref, wf1_ref, bf1_ref, wf2_ref, bf2_ref, out_ref = refs
    h = jnp.maximum(jnp.dot(xf_ref[...], wf1_ref[...],
                            preferred_element_type=jnp.float32)
                    + bf1_ref[...], 0.0)
    out_ref[...] = jnp.dot(h, wf2_ref[...],
                           preferred_element_type=jnp.float32) + bf2_ref[...]


def _head(xf, wf1, bf1, wf2, bf2):
    m, d = xf.shape
    full = lambda s: pl.BlockSpec(s, lambda: (0, 0))
    return pl.pallas_call(
        lambda *refs: _head_body(refs),
        in_specs=[full((m, d)), full(wf1.shape), full((1, wf1.shape[1])),
                  full(wf2.shape), full((1, wf2.shape[1]))],
        out_specs=full((m, wf2.shape[1])),
        out_shape=jax.ShapeDtypeStruct((m, wf2.shape[1]), jnp.float32),
    )(xf, wf1, bf1.reshape(1, -1), wf2, bf2.reshape(1, -1))


def _pad_w(w, half, din):
    if half == din:
        return w
    out = jnp.zeros((2 * half, w.shape[1]), jnp.float32)
    out = out.at[:din].set(w[:din])
    out = out.at[half:half + din].set(w[din:])
    return out


def kernel(x, W1, b1, W2, b2, W3, b3, Wf1, bf1, Wf2, bf2):
    B, N, _ = x.shape
    m = B * N
    xf = jnp.pad(x.reshape(m, 3), ((0, 0), (0, 125)))
    for w, b, din, half in ((W1, b1, 3, 16), (W2, b2, 64, 64),
                            (W3, b3, 64, 64)):
        idx = _topk(xf, din)
        diff = _sc_build_diff(xf, idx.reshape(-1), half)
        h = _edge_matmul(diff, xf, _pad_w(w, half, din), b, half)
        xf = _sc_segmax(h, m)
    out = _head(xf, Wf1, bf1, Wf2, bf2)
    return out.reshape(B, N, 1)
